# baseline (speedup 1.0000x reference)
import functools

import numpy as np
import jax
import jax.numpy as jnp
from jax import lax
from jax.experimental import pallas as pl
from jax.experimental.pallas import tpu as pltpu
from jax.experimental.pallas import tpu_sc as plsc

_D = 64
_NE = 800000
_NW = 32
_K = 320
_NCHUNK = _NE // _K
_NPAIR = 40


def _build_body(e1_ref, e2_ref, ts_ref, t1_ref, t2_ref):
    t1_ref[...] = jnp.dot(e1_ref[...], ts_ref[...],
                          preferred_element_type=jnp.float32,
                          precision=lax.Precision.HIGHEST)
    t2_ref[...] = jnp.dot(e2_ref[...], ts_ref[...],
                          preferred_element_type=jnp.float32,
                          precision=lax.Precision.HIGHEST)


def _onehot_consts():
    e1 = np.zeros((64, 32), np.float32)
    for i in range(60):
        a0, a1, a2 = i // 12, (i // 2) % 6, i % 2
        e1[i, 0 + a0] = 1.0
        e1[i, 5 + a1] = 1.0
        e1[i, 11 + a2] = 1.0
    e2 = np.zeros((64, 32), np.float32)
    for i in range(64):
        a3, a4 = i // 8, i % 8
        e2[i, 13 + a3] = 1.0
        e2[i, 21 + a4] = 1.0
    return e1, e2


_SC_MESH = plsc.VectorSubcoreMesh(core_axis_name="c", subcore_axis_name="s")


@functools.partial(
    pl.kernel,
    out_type=jax.ShapeDtypeStruct((_NE, _D), jnp.float32),
    mesh=_SC_MESH,
    scratch_types=[
        pltpu.VMEM((64, _D), jnp.float32),
        pltpu.VMEM((64, _D), jnp.float32),
        pltpu.VMEM((_K * 5,), jnp.int32),
        pltpu.VMEM((_K * 5,), jnp.int32),
        pltpu.VMEM((_K, _D), jnp.float32),
        pltpu.VMEM((_K, _D), jnp.float32),
        pltpu.SemaphoreType.DMA,
        pltpu.SemaphoreType.DMA,
        pltpu.SemaphoreType.DMA,
        pltpu.SemaphoreType.DMA,
    ],
)
def _sc_embed(t1_hbm, t2_hbm, ea_hbm, out_hbm,
              t1v, t2v, ea0v, ea1v, o0v, o1v, sea0, sea1, so0, so1):
    wid = lax.axis_index("s") * 2 + lax.axis_index("c")
    pltpu.sync_copy(t1_hbm, t1v)
    pltpu.sync_copy(t2_hbm, t2v)
    eav = (ea0v, ea1v)
    outv = (o0v, o1v)
    sea = (sea0, sea1)
    so = (so0, so1)

    pltpu.async_copy(ea_hbm.at[pl.ds(wid * _K * 5, _K * 5)], ea0v, sea0)

    def compute_chunk(eab, outb):
        @plsc.parallel_loop(0, _K // 16, unroll=2)
        def group(g):
            o = g * 16
            w = [eab[pl.ds(o * 5 + k * 16, 16)] for k in range(5)]

            def feat(l, t):
                p = 5 * l + t
                return w[p // 16][p % 16]

            for l in range(16):
                s1 = feat(l, 0) * 12 + feat(l, 1) * 2 + feat(l, 2)
                s2 = feat(l, 3) * 8 + feat(l, 4)
                for c in range(4):
                    v = (t1v[s1, pl.ds(c * 16, 16)]
                         + t2v[s2, pl.ds(c * 16, 16)])
                    outv_row = o + l
                    outb[outv_row, pl.ds(c * 16, 16)] = v

    def pair(i2, carry):
        for b in (0, 1):
            j = 2 * i2 + b
            c = wid + j * _NW

            @pl.when(c < _NCHUNK)
            def _():
                pltpu.make_async_copy(
                    ea_hbm.at[pl.ds(c * _K * 5, _K * 5)], eav[b], sea[b]
                ).wait()
                @pl.when(c + _NW < _NCHUNK)
                def _():
                    pltpu.async_copy(
                        ea_hbm.at[pl.ds((c + _NW) * _K * 5, _K * 5)],
                        eav[1 - b], sea[1 - b])
                @pl.when(j >= 2)
                def _():
                    pltpu.make_async_copy(
                        outv[b], out_hbm.at[pl.ds(0, _K)], so[b]).wait()
                compute_chunk(eav[b], outv[b])
                pltpu.async_copy(outv[b], out_hbm.at[pl.ds(c * _K, _K)],
                                 so[b])
        return carry

    lax.fori_loop(0, _NPAIR, pair, 0)
    pltpu.make_async_copy(o0v, out_hbm.at[pl.ds(0, _K)], so0).wait()
    pltpu.make_async_copy(o1v, out_hbm.at[pl.ds(0, _K)], so1).wait()


def kernel(edge_attr, table_0, table_1, table_2, table_3, table_4):
    stacked = jnp.concatenate(
        [table_0, table_1, table_2, table_3, table_4,
         jnp.zeros((3, _D), jnp.float32)], axis=0)
    e1c, e2c = _onehot_consts()
    t1, t2 = pl.pallas_call(
        _build_body,
        out_shape=(jax.ShapeDtypeStruct((64, _D), jnp.float32),
                   jax.ShapeDtypeStruct((64, _D), jnp.float32)),
    )(jnp.asarray(e1c), jnp.asarray(e2c), stacked)
    ea_flat = edge_attr.reshape(_NE * 5)
    return _sc_embed(t1, t2, ea_flat)
